# baseline (speedup 1.0000x reference)
import jax
import jax.numpy as jnp
from jax.experimental import pallas as pl
from jax.experimental.pallas import tpu as pltpu

N = 8
DEPTH = 3
DIM = 1024
BLK = 512

PREC = jax.lax.Precision.DEFAULT


def _nt(a, b):
    return jax.lax.dot_general(a, b, (((1,), (1,)), ((), ())),
                               preferred_element_type=jnp.float32,
                               precision=PREC)


def _nn(a, b):
    return jax.lax.dot_general(a, b, (((1,), (0,)), ((), ())),
                               preferred_element_type=jnp.float32,
                               precision=PREC)


def _tree_kernel(x_ref, w0_ref, w1_ref, w2_ref, v0_ref, v1_ref, v2_ref,
                 ws_ref, out_ref):
    f32 = jnp.float32
    xb = x_ref[...]
    T = xb.shape[0]
    NEG = f32(-1e30)
    cols8 = jax.lax.broadcasted_iota(jnp.int32, (T, N), 1)

    a0 = _nt(xb, w0_ref[...])
    m0 = jnp.max(a0, axis=1, keepdims=True)
    k0 = jnp.min(jnp.where(a0 == m0, cols8, N), axis=1, keepdims=True)
    oh0 = (cols8 == k0).astype(f32)
    out_ref[0, :, 0, :] = _nn(oh0, v0_ref[...])
    x1 = xb + _nn(oh0, ws_ref[...])

    a1 = _nt(x1, w1_ref[...])
    cols64 = jax.lax.broadcasted_iota(jnp.int32, (T, N * N), 1)
    a1m = jnp.where((cols64 >> 3) == k0, a1, NEG)
    m1 = jnp.max(a1m, axis=1, keepdims=True)
    c1 = jnp.min(jnp.where(a1m == m1, cols64, N * N), axis=1, keepdims=True)
    oh1 = (cols64 == c1).astype(f32)
    out_ref[1, :, 0, :] = _nn(oh1, v1_ref[...])
    x2 = x1 + _nn((cols8 == (c1 & 7)).astype(f32), ws_ref[...])

    a2 = _nt(x2, w2_ref[...])
    cols512 = jax.lax.broadcasted_iota(jnp.int32, (T, N * N * N), 1)
    a2m = jnp.where((cols512 >> 3) == c1, a2, NEG)
    m2 = jnp.max(a2m, axis=1, keepdims=True)
    c2 = jnp.min(jnp.where(a2m == m2, cols512, N * N * N), axis=1,
                 keepdims=True)
    oh2 = (cols512 == c2).astype(f32)
    out_ref[2, :, 0, :] = _nn(oh2, v2_ref[...])


def kernel(x, level0_data, level0_values, level1_data, level1_values,
           level2_data, level2_values, W_state):
    B = x.shape[0]
    w0 = level0_data.reshape(N, DIM)
    w1 = level1_data.reshape(N * N, DIM)
    w2 = level2_data.reshape(N * N * N, DIM)
    v0 = level0_values.reshape(N, DIM)
    v1 = level1_values.reshape(N * N, DIM)
    v2 = level2_values.reshape(N * N * N, DIM)

    grid = (B // BLK,)
    full = lambda shape: pl.BlockSpec(shape, lambda i: (0,) * len(shape))
    out = pl.pallas_call(
        _tree_kernel,
        grid=grid,
        in_specs=[
            pl.BlockSpec((BLK, DIM), lambda i: (i, 0)),
            full((N, DIM)), full((N * N, DIM)), full((N * N * N, DIM)),
            full((N, DIM)), full((N * N, DIM)), full((N * N * N, DIM)),
            full((N, DIM)),
        ],
        out_specs=pl.BlockSpec((DEPTH, BLK, 1, DIM), lambda i: (0, i, 0, 0)),
        out_shape=jax.ShapeDtypeStruct((DEPTH, B, 1, DIM), jnp.float32),
        compiler_params=pltpu.CompilerParams(dimension_semantics=("parallel",)),
    )(x, w0, w1, w2, v0, v1, v2, W_state)
    return out


# speedup vs baseline: 1.0105x; 1.0105x over previous
import jax
import jax.numpy as jnp
from jax.experimental import pallas as pl
from jax.experimental.pallas import tpu as pltpu

N = 8
DEPTH = 3
DIM = 1024
BLK = 1024

PREC = jax.lax.Precision.DEFAULT


def _nt(a, b):
    return jax.lax.dot_general(a, b, (((1,), (1,)), ((), ())),
                               preferred_element_type=jnp.float32,
                               precision=PREC)


def _nn(a, b):
    return jax.lax.dot_general(a, b, (((1,), (0,)), ((), ())),
                               preferred_element_type=jnp.float32,
                               precision=PREC)


def _tree_kernel(x_ref, w0_ref, w1_ref, w2_ref, v0_ref, v1_ref, v2_ref,
                 ws_ref, out_ref, xcur_ref, cprev_ref):
    f32 = jnp.float32
    d = pl.program_id(1)
    T = x_ref.shape[0]
    NEG = f32(-1e30)
    cols8 = jax.lax.broadcasted_iota(jnp.int32, (T, N), 1)

    @pl.when(d == 0)
    def _level0():
        xb = x_ref[...]
        a0 = _nt(xb, w0_ref[...])
        m0 = jnp.max(a0, axis=1, keepdims=True)
        k0 = jnp.min(jnp.where(a0 == m0, cols8, N), axis=1, keepdims=True)
        oh0 = (cols8 == k0).astype(f32)
        out_ref[0, :, 0, :] = _nn(oh0, v0_ref[...])
        xcur_ref[...] = xb + _nn(oh0, ws_ref[...])
        cprev_ref[...] = k0

    @pl.when(d == 1)
    def _level1():
        x1 = xcur_ref[...]
        k0 = cprev_ref[...]
        a1 = _nt(x1, w1_ref[...])
        cols64 = jax.lax.broadcasted_iota(jnp.int32, (T, N * N), 1)
        a1m = jnp.where((cols64 >> 3) == k0, a1, NEG)
        m1 = jnp.max(a1m, axis=1, keepdims=True)
        c1 = jnp.min(jnp.where(a1m == m1, cols64, N * N), axis=1,
                     keepdims=True)
        oh1 = (cols64 == c1).astype(f32)
        out_ref[0, :, 0, :] = _nn(oh1, v1_ref[...])
        xcur_ref[...] = x1 + _nn((cols8 == (c1 & 7)).astype(f32),
                                 ws_ref[...])
        cprev_ref[...] = c1

    @pl.when(d == 2)
    def _level2():
        x2 = xcur_ref[...]
        c1 = cprev_ref[...]
        a2 = _nt(x2, w2_ref[...])
        cols512 = jax.lax.broadcasted_iota(jnp.int32, (T, N * N * N), 1)
        a2m = jnp.where((cols512 >> 3) == c1, a2, NEG)
        m2 = jnp.max(a2m, axis=1, keepdims=True)
        c2 = jnp.min(jnp.where(a2m == m2, cols512, N * N * N), axis=1,
                     keepdims=True)
        oh2 = (cols512 == c2).astype(f32)
        out_ref[0, :, 0, :] = _nn(oh2, v2_ref[...])


def kernel(x, level0_data, level0_values, level1_data, level1_values,
           level2_data, level2_values, W_state):
    B = x.shape[0]
    w0 = level0_data.reshape(N, DIM)
    w1 = level1_data.reshape(N * N, DIM)
    w2 = level2_data.reshape(N * N * N, DIM)
    v0 = level0_values.reshape(N, DIM)
    v1 = level1_values.reshape(N * N, DIM)
    v2 = level2_values.reshape(N * N * N, DIM)

    grid = (B // BLK, DEPTH)
    full = lambda shape: pl.BlockSpec(shape, lambda i, d: (0,) * len(shape))
    out = pl.pallas_call(
        _tree_kernel,
        grid=grid,
        in_specs=[
            pl.BlockSpec((BLK, DIM), lambda i, d: (i, 0)),
            full((N, DIM)), full((N * N, DIM)), full((N * N * N, DIM)),
            full((N, DIM)), full((N * N, DIM)), full((N * N * N, DIM)),
            full((N, DIM)),
        ],
        out_specs=pl.BlockSpec((1, BLK, 1, DIM), lambda i, d: (d, i, 0, 0)),
        out_shape=jax.ShapeDtypeStruct((DEPTH, B, 1, DIM), jnp.float32),
        scratch_shapes=[
            pltpu.VMEM((BLK, DIM), jnp.float32),
            pltpu.VMEM((BLK, 1), jnp.int32),
        ],
    )(x, w0, w1, w2, v0, v1, v2, W_state)
    return out


# speedup vs baseline: 1.0809x; 1.0697x over previous
import jax
import jax.numpy as jnp
from jax.experimental import pallas as pl
from jax.experimental.pallas import tpu as pltpu

N = 8
DEPTH = 3
DIM = 1024
BLK = 1024

PREC = jax.lax.Precision.DEFAULT


def _nt(a, b):
    return jax.lax.dot_general(a, b, (((1,), (1,)), ((), ())),
                               preferred_element_type=jnp.float32,
                               precision=PREC)


def _nn(a, b):
    return jax.lax.dot_general(a, b, (((1,), (0,)), ((), ())),
                               preferred_element_type=jnp.float32,
                               precision=PREC)


def _tree_kernel(x_ref, w0_ref, w1_ref, w2_ref, v0_ref, v1_ref, v2_ref,
                 ws_ref, out_ref):
    f32 = jnp.float32
    xb = x_ref[...]
    T = xb.shape[0]
    NEG = f32(-1e30)
    cols8 = jax.lax.broadcasted_iota(jnp.int32, (T, N), 1)

    a0 = _nt(xb, w0_ref[...])
    m0 = jnp.max(a0, axis=1, keepdims=True)
    k0 = jnp.min(jnp.where(a0 == m0, cols8, N), axis=1, keepdims=True)
    oh0 = (cols8 == k0).astype(f32)
    out_ref[0, :, 0, :] = _nn(oh0, v0_ref[...])
    x1 = xb + _nn(oh0, ws_ref[...])

    a1 = _nt(x1, w1_ref[...])
    cols64 = jax.lax.broadcasted_iota(jnp.int32, (T, N * N), 1)
    a1m = jnp.where((cols64 >> 3) == k0, a1, NEG)
    m1 = jnp.max(a1m, axis=1, keepdims=True)
    c1 = jnp.min(jnp.where(a1m == m1, cols64, N * N), axis=1, keepdims=True)
    oh1 = (cols64 == c1).astype(f32)
    out_ref[1, :, 0, :] = _nn(oh1, v1_ref[...])
    x2 = x1 + _nn((cols8 == (c1 & 7)).astype(f32), ws_ref[...])

    a2 = _nt(x2, w2_ref[...])
    cols512 = jax.lax.broadcasted_iota(jnp.int32, (T, N * N * N), 1)
    a2m = jnp.where((cols512 >> 3) == c1, a2, NEG)
    m2 = jnp.max(a2m, axis=1, keepdims=True)
    c2 = jnp.min(jnp.where(a2m == m2, cols512, N * N * N), axis=1,
                 keepdims=True)
    oh2 = (cols512 == c2).astype(f32)
    out_ref[2, :, 0, :] = _nn(oh2, v2_ref[...])


def kernel(x, level0_data, level0_values, level1_data, level1_values,
           level2_data, level2_values, W_state):
    B = x.shape[0]
    w0 = level0_data.reshape(N, DIM)
    w1 = level1_data.reshape(N * N, DIM)
    w2 = level2_data.reshape(N * N * N, DIM)
    v0 = level0_values.reshape(N, DIM)
    v1 = level1_values.reshape(N * N, DIM)
    v2 = level2_values.reshape(N * N * N, DIM)

    grid = (B // BLK,)
    full = lambda shape: pl.BlockSpec(shape, lambda i: (0,) * len(shape))
    out = pl.pallas_call(
        _tree_kernel,
        grid=grid,
        in_specs=[
            pl.BlockSpec((BLK, DIM), lambda i: (i, 0)),
            full((N, DIM)), full((N * N, DIM)), full((N * N * N, DIM)),
            full((N, DIM)), full((N * N, DIM)), full((N * N * N, DIM)),
            full((N, DIM)),
        ],
        out_specs=pl.BlockSpec((DEPTH, BLK, 1, DIM), lambda i: (0, i, 0, 0)),
        out_shape=jax.ShapeDtypeStruct((DEPTH, B, 1, DIM), jnp.float32),
        compiler_params=pltpu.CompilerParams(dimension_semantics=("parallel",)),
    )(x, w0, w1, w2, v0, v1, v2, W_state)
    return out
